# baseline (speedup 1.0000x reference)
import jax
import jax.numpy as jnp
from jax.experimental import pallas as pl
from jax.experimental.pallas import tpu as pltpu
from jax.experimental.pallas import tpu_sc as plsc

_NUM_SC = 2
_CHUNK_ROWS = 128
_NBUF = 4


def _sc_body_factory(batch, pos, d):
    rows_per_core = pos // _NUM_SC
    nchunks = rows_per_core // _CHUNK_ROWS

    def body(w_hbm, o_hbm, buf, sem_in, sem_out):
        core = jax.lax.axis_index("core")
        base = core * rows_per_core

        ins = []
        outs = []
        for i in range(min(_NBUF, nchunks)):
            c = pltpu.async_copy(
                w_hbm.at[pl.ds(base + i * _CHUNK_ROWS, _CHUNK_ROWS), :],
                buf.at[i % _NBUF],
                sem_in.at[i % _NBUF],
            )
            ins.append(c)
        for i in range(nchunks):
            b = i % _NBUF
            ins[i].wait()
            chunk_outs = []
            for bb in range(batch):
                c = pltpu.async_copy(
                    buf.at[b],
                    o_hbm.at[bb, pl.ds(base + i * _CHUNK_ROWS, _CHUNK_ROWS), :],
                    sem_out.at[b, bb],
                )
                chunk_outs.append(c)
            outs.append(chunk_outs)
            j = i + _NBUF
            if j < nchunks:
                for c in outs[i]:
                    c.wait()
                nc = pltpu.async_copy(
                    w_hbm.at[pl.ds(base + j * _CHUNK_ROWS, _CHUNK_ROWS), :],
                    buf.at[j % _NBUF],
                    sem_in.at[j % _NBUF],
                )
                ins.append(nc)
        for i in range(max(0, nchunks - _NBUF), nchunks):
            for c in outs[i]:
                c.wait()

    return body


def kernel(tokens, W_pos):
    batch, pos = tokens.shape
    n_ctx, d = W_pos.shape
    mesh = plsc.ScalarSubcoreMesh(axis_name="core", num_cores=_NUM_SC)
    body = _sc_body_factory(batch, pos, d)
    k = pl.kernel(
        body,
        out_type=jax.ShapeDtypeStruct((batch, pos, d), W_pos.dtype),
        mesh=mesh,
        scratch_types=[
            pltpu.VMEM_SHARED((_NBUF, _CHUNK_ROWS, d), W_pos.dtype),
            pltpu.SemaphoreType.DMA((_NBUF,)),
            pltpu.SemaphoreType.DMA((_NBUF, 2)),
        ],
    )
    return k(W_pos)


# speedup vs baseline: 2.2027x; 2.2027x over previous
import jax
import jax.numpy as jnp
from jax.experimental import pallas as pl
from jax.experimental.pallas import tpu as pltpu

_CHUNK_ROWS = (64, 64, 128, 256, 512, 512, 512, 512, 512, 512, 512)


def _make_body(batch, pos, d):
    starts = []
    off = 0
    for r in _CHUNK_ROWS:
        starts.append(off)
        off += r
    assert off == pos

    def body(w_hbm, o_hbm, vmem, sem_in, sem_out):
        ins = []
        for i, (s, r) in enumerate(zip(starts, _CHUNK_ROWS)):
            c = pltpu.make_async_copy(
                w_hbm.at[pl.ds(s, r), :],
                vmem.at[pl.ds(s, r), :],
                sem_in.at[i],
            )
            c.start()
            ins.append(c)
        outs = []
        for i, (s, r) in enumerate(zip(starts, _CHUNK_ROWS)):
            ins[i].wait()
            for b in range(batch):
                c = pltpu.make_async_copy(
                    vmem.at[pl.ds(s, r), :],
                    o_hbm.at[b, pl.ds(s, r), :],
                    sem_out.at[i, b],
                )
                c.start()
                outs.append(c)
        for c in outs:
            c.wait()

    return body


def kernel(tokens, W_pos):
    batch, pos = tokens.shape
    n_ctx, d = W_pos.shape
    n = len(_CHUNK_ROWS)
    out = pl.pallas_call(
        _make_body(batch, pos, d),
        in_specs=[pl.BlockSpec(memory_space=pl.ANY)],
        out_specs=pl.BlockSpec(memory_space=pl.ANY),
        out_shape=jax.ShapeDtypeStruct((batch, pos, d), W_pos.dtype),
        scratch_shapes=[
            pltpu.VMEM((pos, d), W_pos.dtype),
            pltpu.SemaphoreType.DMA((n,)),
            pltpu.SemaphoreType.DMA((n, 2)),
        ],
    )(W_pos)
    return out
